# baseline (speedup 1.0000x reference)
import functools

import jax
import jax.numpy as jnp
from jax import lax
from jax.experimental import pallas as pl
from jax.experimental.pallas import tpu as pltpu
from jax.experimental.pallas import tpu_sc as plsc

N = 10000
D = 128
DA = 144
E = 320000
NC, NS = 2, 16
NW = NC * NS
CH = 128
CPW0 = 111
CPW1 = 47
CPWM = max(CPW0, CPW1)
E0 = NS * CPW0 * CH
E1 = NS * CPW1 * CH
RPT = 640
NP = NS * RPT


def _sc_aggregate(xa, src3, dst3, zeros):
  mesh = plsc.VectorSubcoreMesh(
      core_axis_name="c", subcore_axis_name="s",
      num_cores=NC, num_subcores=NS)

  @functools.partial(
      pl.kernel,
      name="rgcn_sc_aggregate",
      out_type=jax.ShapeDtypeStruct((NC * NP, DA), jnp.float32),
      mesh=mesh,
      compiler_params=pltpu.CompilerParams(use_tc_tiling_on_sc=False),
      scratch_types=[
          pltpu.VMEM((CH,), jnp.int32),
          pltpu.VMEM((CH,), jnp.int32),
          pltpu.VMEM((CH,), jnp.int32),
          pltpu.VMEM((CH,), jnp.int32),
          pltpu.VMEM((CH, DA), jnp.float32),
          pltpu.VMEM((CH, DA), jnp.float32),
          pltpu.VMEM_SHARED((NP, DA), jnp.float32),
          pltpu.SemaphoreType.DMA,
          pltpu.SemaphoreType.DMA,
          pltpu.SemaphoreType.DMA,
          pltpu.SemaphoreType.DMA,
          pltpu.SemaphoreType.DMA,
          pltpu.SemaphoreType.DMA,
      ],
  )
  def body(xa_hbm, src_hbm, dst_hbm, zeros_hbm, out_hbm,
           srcb0, srcb1, dstb0, dstb1, rows0_v, rows1_v, acc_sh,
           semg0, semg1, semsi0, semsi1, semd0, semd1):
    cid = lax.axis_index("c")
    sid = lax.axis_index("s")
    wid = cid * NS + sid
    base = sid * RPT
    cpw = jnp.where(cid == 0, CPW0, CPW1)
    last = cpw - 1

    def wait_rows(buf_v, sem):
      pltpu.make_async_copy(zeros_hbm.at[pl.ds(0, CH)], buf_v, sem).wait()

    def wait_idx(buf_v, sem):
      pltpu.make_async_copy(src_hbm.at[0, 0], buf_v, sem).wait()

    pltpu.async_copy(src_hbm.at[wid, 0], srcb0, semsi0)
    pltpu.async_copy(dst_hbm.at[wid, 0], dstb0, semd0)
    pltpu.async_copy(src_hbm.at[wid, 1], srcb1, semsi1)
    pltpu.async_copy(dst_hbm.at[wid, 1], dstb1, semd1)
    pltpu.sync_copy(zeros_hbm.at[pl.ds(base, RPT)],
                    acc_sh.at[pl.ds(base, RPT)])
    plsc.subcore_barrier()

    wait_idx(srcb0, semsi0)
    pltpu.async_copy(xa_hbm.at[srcb0], rows0_v, semg0)

    def pair(g, carry):
      a = 2 * g
      wait_idx(srcb1, semsi1)
      pltpu.async_copy(xa_hbm.at[srcb1], rows1_v, semg1)
      wait_rows(rows0_v, semg0)
      pltpu.async_copy(src_hbm.at[wid, jnp.minimum(a + 2, last)],
                       srcb0, semsi0)
      wait_idx(dstb0, semd0)
      pltpu.sync_copy(rows0_v, acc_sh.at[dstb0], add=True)
      pltpu.async_copy(dst_hbm.at[wid, jnp.minimum(a + 2, last)],
                       dstb0, semd0)
      wait_idx(srcb0, semsi0)
      pltpu.async_copy(xa_hbm.at[srcb0], rows0_v, semg0)
      wait_rows(rows1_v, semg1)
      pltpu.async_copy(src_hbm.at[wid, jnp.minimum(a + 3, last)],
                       srcb1, semsi1)
      wait_idx(dstb1, semd1)
      pltpu.sync_copy(rows1_v, acc_sh.at[dstb1], add=True)
      pltpu.async_copy(dst_hbm.at[wid, jnp.minimum(a + 3, last)],
                       dstb1, semd1)
      return carry

    lax.fori_loop(0, (cpw - 1) // 2, pair, 0)
    wait_rows(rows0_v, semg0)
    wait_idx(dstb0, semd0)
    pltpu.sync_copy(rows0_v, acc_sh.at[dstb0], add=True)
    wait_idx(srcb1, semsi1)
    wait_idx(dstb1, semd1)
    plsc.subcore_barrier()
    pltpu.sync_copy(acc_sh.at[pl.ds(base, RPT)],
                    out_hbm.at[pl.ds(cid * NP + base, RPT)])

  return body(xa, src3, dst3, zeros)


def _tc_combine(x, p0, p1, wrT, wlT, b):
  BLK = 2000

  def body(x_ref, p0_ref, p1_ref, wr_ref, wl_ref, b_ref, o_ref):
    msum = p0_ref[:, :D] + p1_ref[:, :D]
    cnt = p0_ref[:, D:D + 1] + p1_ref[:, D:D + 1]
    agg = msum * (1.0 / jnp.maximum(cnt, 1.0))
    o_ref[...] = (
        jnp.dot(x_ref[...], wr_ref[...], preferred_element_type=jnp.float32)
        + jnp.dot(agg, wl_ref[...], preferred_element_type=jnp.float32)
        + b_ref[...])

  return pl.pallas_call(
      body,
      grid=(N // BLK,),
      in_specs=[
          pl.BlockSpec((BLK, D), lambda i: (i, 0)),
          pl.BlockSpec((BLK, DA), lambda i: (i, 0)),
          pl.BlockSpec((BLK, DA), lambda i: (i, 0)),
          pl.BlockSpec((D, D), lambda i: (0, 0)),
          pl.BlockSpec((D, D), lambda i: (0, 0)),
          pl.BlockSpec((1, D), lambda i: (0, 0)),
      ],
      out_specs=pl.BlockSpec((BLK, D), lambda i: (i, 0)),
      out_shape=jax.ShapeDtypeStruct((N, D), jnp.float32),
  )(x, p0, p1, wrT, wlT, b)


def kernel(x, edge_index, W_rel, W_root, b_root):
  xa = jnp.concatenate(
      [x, jnp.ones((N, 1), jnp.float32), jnp.zeros((N, DA - D - 1), jnp.float32)],
      axis=1)
  pad = E0 + E1 - E
  src = jnp.concatenate([edge_index[0], jnp.zeros((pad,), jnp.int32)])
  dst = jnp.concatenate([edge_index[1], jnp.full((pad,), N, jnp.int32)])

  def split3(a, fill):
    a0 = a[:E0].reshape(NS, CPW0, CH)
    a1 = a[E0:].reshape(NS, CPW1, CH)
    a0 = jnp.pad(a0, ((0, 0), (0, CPWM - CPW0), (0, 0)), constant_values=fill)
    a1 = jnp.pad(a1, ((0, 0), (0, CPWM - CPW1), (0, 0)), constant_values=fill)
    return jnp.concatenate([a0, a1], axis=0)

  src3 = split3(src, 0)
  dst3 = split3(dst, N)
  zeros = jnp.zeros((NP, DA), jnp.float32)

  parts = _sc_aggregate(xa, src3, dst3, zeros)
  return _tc_combine(x, parts[:N], parts[NP:NP + N], W_root.T, W_rel.T,
                     b_root.reshape(1, D))


# speedup vs baseline: 1.2397x; 1.2397x over previous
import functools

import jax
import jax.numpy as jnp
from jax import lax
from jax.experimental import pallas as pl
from jax.experimental.pallas import tpu as pltpu
from jax.experimental.pallas import tpu_sc as plsc

N = 10000
D = 128
DA = 144
E = 320000
NC, NS = 2, 16
NW = NC * NS
NBUF = 3
CH = 80
CPW = 126
EP = NW * CPW * CH
RPT = 640
NP = NS * RPT


def _sc_aggregate(xa, src3, dst3, zeros):
  mesh = plsc.VectorSubcoreMesh(
      core_axis_name="c", subcore_axis_name="s",
      num_cores=NC, num_subcores=NS)

  @functools.partial(
      pl.kernel,
      name="rgcn_sc_aggregate",
      out_type=jax.ShapeDtypeStruct((NC * NP, DA), jnp.float32),
      mesh=mesh,
      compiler_params=pltpu.CompilerParams(use_tc_tiling_on_sc=False),
      scratch_types=[
          [pltpu.VMEM((CH,), jnp.int32)] * NBUF,
          [pltpu.VMEM((CH,), jnp.int32)] * NBUF,
          [pltpu.VMEM((CH, DA), jnp.float32)] * NBUF,
          pltpu.VMEM_SHARED((NP, DA), jnp.float32),
          [pltpu.SemaphoreType.DMA] * NBUF,
          [pltpu.SemaphoreType.DMA] * NBUF,
          [pltpu.SemaphoreType.DMA] * NBUF,
      ],
  )
  def body(xa_hbm, src_hbm, dst_hbm, zeros_hbm, out_hbm,
           srcbs, dstbs, rows, acc_sh, semg, semsi, semd):
    cid = lax.axis_index("c")
    sid = lax.axis_index("s")
    wid = cid * NS + sid
    base = sid * RPT

    def wait_rows(buf_v, sem):
      pltpu.make_async_copy(zeros_hbm.at[pl.ds(0, CH)], buf_v, sem).wait()

    def wait_idx(buf_v, sem):
      pltpu.make_async_copy(src_hbm.at[0, 0], buf_v, sem).wait()

    for k in range(NBUF):
      pltpu.async_copy(src_hbm.at[wid, k], srcbs[k], semsi[k])
      pltpu.async_copy(dst_hbm.at[wid, k], dstbs[k], semd[k])
    pltpu.sync_copy(zeros_hbm.at[pl.ds(base, RPT)],
                    acc_sh.at[pl.ds(base, RPT)])
    plsc.subcore_barrier()
    for k in range(NBUF):
      wait_idx(srcbs[k], semsi[k])
      pltpu.async_copy(xa_hbm.at[srcbs[k]], rows[k], semg[k])

    def rotation(g, carry):
      a0 = NBUF * g
      for k in range(NBUF):
        a = a0 + k
        wait_rows(rows[k], semg[k])
        pltpu.async_copy(src_hbm.at[wid, a + NBUF], srcbs[k], semsi[k])
        wait_idx(dstbs[k], semd[k])
        pltpu.sync_copy(rows[k], acc_sh.at[dstbs[k]], add=True)
        pltpu.async_copy(dst_hbm.at[wid, a + NBUF], dstbs[k], semd[k])
        wait_idx(srcbs[k], semsi[k])
        pltpu.async_copy(xa_hbm.at[srcbs[k]], rows[k], semg[k])
      return carry

    lax.fori_loop(0, CPW // NBUF - 1, rotation, 0)
    for k in range(NBUF):
      wait_rows(rows[k], semg[k])
      wait_idx(dstbs[k], semd[k])
      pltpu.sync_copy(rows[k], acc_sh.at[dstbs[k]], add=True)
    plsc.subcore_barrier()
    pltpu.sync_copy(acc_sh.at[pl.ds(base, RPT)],
                    out_hbm.at[pl.ds(cid * NP + base, RPT)])

  return body(xa, src3, dst3, zeros)


def _tc_combine(x, p0, p1, wrT, wlT, b):
  BLK = 2000

  def body(x_ref, p0_ref, p1_ref, wr_ref, wl_ref, b_ref, o_ref):
    msum = p0_ref[:, :D] + p1_ref[:, :D]
    cnt = p0_ref[:, D:D + 1] + p1_ref[:, D:D + 1]
    agg = msum * (1.0 / jnp.maximum(cnt, 1.0))
    o_ref[...] = (
        jnp.dot(x_ref[...], wr_ref[...], preferred_element_type=jnp.float32)
        + jnp.dot(agg, wl_ref[...], preferred_element_type=jnp.float32)
        + b_ref[...])

  return pl.pallas_call(
      body,
      grid=(N // BLK,),
      in_specs=[
          pl.BlockSpec((BLK, D), lambda i: (i, 0)),
          pl.BlockSpec((BLK, DA), lambda i: (i, 0)),
          pl.BlockSpec((BLK, DA), lambda i: (i, 0)),
          pl.BlockSpec((D, D), lambda i: (0, 0)),
          pl.BlockSpec((D, D), lambda i: (0, 0)),
          pl.BlockSpec((1, D), lambda i: (0, 0)),
      ],
      out_specs=pl.BlockSpec((BLK, D), lambda i: (i, 0)),
      out_shape=jax.ShapeDtypeStruct((N, D), jnp.float32),
  )(x, p0, p1, wrT, wlT, b)


def kernel(x, edge_index, W_rel, W_root, b_root):
  xa = jnp.concatenate(
      [x, jnp.ones((N, 1), jnp.float32), jnp.zeros((N, DA - D - 1), jnp.float32)],
      axis=1)
  pad = EP - E
  src = jnp.concatenate([edge_index[0], jnp.zeros((pad,), jnp.int32)])
  dst = jnp.concatenate([edge_index[1], jnp.full((pad,), N, jnp.int32)])
  src3 = src.reshape(NW, CPW, CH)
  dst3 = dst.reshape(NW, CPW, CH)
  zeros = jnp.zeros((NP, DA), jnp.float32)

  parts = _sc_aggregate(xa, src3, dst3, zeros)
  return _tc_combine(x, parts[:N], parts[NP:NP + N], W_root.T, W_rel.T,
                     b_root.reshape(1, D))
